# baseline (speedup 1.0000x reference)
import functools

import jax
import jax.numpy as jnp
from jax import lax
from jax.experimental import pallas as pl
from jax.experimental.pallas import tpu as pltpu
from jax.experimental.pallas import tpu_sc as plsc

_IGNORE = -100
_LAMBDA = 0.5
_R = 2048


def _tail_part(h, w2, b2, t, keep):
    logits = jnp.dot(h, w2, preferred_element_type=jnp.float32) + b2
    C = logits.shape[1]

    m = jnp.max(logits, axis=1)
    lse = m + jnp.log(jnp.sum(jnp.exp(logits - m[:, None]), axis=1))
    iota_c = lax.broadcasted_iota(jnp.int32, logits.shape, 1)
    xt = jnp.sum(jnp.where(iota_c == t[:, None], logits, 0.0), axis=1)
    validf = (t != _IGNORE).astype(jnp.float32)
    nll_sum = jnp.sum((lse - xt) * validf)
    valid_cnt = jnp.sum(validf)

    pred = jnp.min(jnp.where(logits == m[:, None], iota_c, C), axis=1)
    keepb = keep > 0
    correct = jnp.sum(((pred == t) & keepb).astype(jnp.float32))
    mvalid_cnt = jnp.sum(keepb.astype(jnp.float32))

    rows = lax.broadcasted_iota(jnp.int32, (8, 128), 0)
    cols = lax.broadcasted_iota(jnp.int32, (8, 128), 1)
    r0 = rows == 0
    return (jnp.where(r0 & (cols == 0), nll_sum, 0.0)
            + jnp.where(r0 & (cols == 1), valid_cnt, 0.0)
            + jnp.where(r0 & (cols == 2), correct, 0.0)
            + jnp.where(r0 & (cols == 3), mvalid_cnt, 0.0))


def _tc_body(x_ref, w1_ref, b1_ref, w2_ref, b2_ref, st_ref, mv_ref,
             stc_ref, mvc_ref, out_ref, h_ref):
    i = pl.program_id(0)
    G = pl.num_programs(0)

    @pl.when(i == 0)
    def _():
        h_ref[...] = jnp.zeros_like(h_ref)
        out_ref[...] = jnp.zeros_like(out_ref)

    part = _tail_part(h_ref[...], w2_ref[...], b2_ref[...],
                      st_ref[0, 0, :], mv_ref[0, 0, :])
    gate = jnp.where(i > 0, 1.0, 0.0).astype(jnp.float32)
    out_ref[...] += part * gate

    h_ref[...] = jnp.tanh(
        jnp.dot(x_ref[...], w1_ref[...], preferred_element_type=jnp.float32)
        + b1_ref[...])

    @pl.when(i == G - 1)
    def _():
        out_ref[...] += _tail_part(h_ref[...], w2_ref[...], b2_ref[...],
                                   stc_ref[0, 0, :], mvc_ref[0, 0, :])


def _make_sc_realign(B, S):
    L = 16
    NCH = S // L
    mesh = plsc.VectorSubcoreMesh(core_axis_name="c", subcore_axis_name="s")

    @functools.partial(
        pl.kernel, mesh=mesh,
        compiler_params=pltpu.CompilerParams(needs_layout_passes=False),
        out_type=(
            jax.ShapeDtypeStruct((B, S), jnp.int32),
            jax.ShapeDtypeStruct((B, S), jnp.int32),
            jax.ShapeDtypeStruct((B * S,), jnp.int32),
            jax.ShapeDtypeStruct((B, L), jnp.int32),
            jax.ShapeDtypeStruct((B, L), jnp.int32),
        ),
        scratch_types=[
            pltpu.VMEM((S,), jnp.int32),
            pltpu.VMEM((S,), jnp.int32),
            pltpu.VMEM((S,), jnp.int32),
            pltpu.VMEM((S,), jnp.int32),
            pltpu.VMEM((S,), jnp.int32),
            pltpu.VMEM((S,), jnp.int32),
            pltpu.VMEM((S + L,), jnp.int32),
            pltpu.VMEM((B * S,), jnp.int32),
            pltpu.VMEM((B, L), jnp.int32),
            pltpu.VMEM((B, L), jnp.int32),
            pltpu.VMEM((L,), jnp.int32),
            pltpu.VMEM((L,), jnp.int32),
        ])
    def realign(tm_hbm, tmm_hbm, tag_hbm, tagm_hbm,
                sp_hbm, keep_hbm, v_hbm, cnt_hbm, ts_hbm,
                tmm_v, tok_v, tag_v, tagm_v, keep_v, tokcs_v, tagbuf,
                vbuf, clocal, tslocal, offarr, stg):
        c = lax.axis_index("c")
        b = lax.axis_index("s")

        @pl.when(c == 0)
        def _phase1():
            pltpu.sync_copy(tmm_hbm.at[b], tmm_v)
            pltpu.sync_copy(tm_hbm.at[b], tok_v)
            pltpu.sync_copy(tag_hbm.at[b], tag_v)
            pltpu.sync_copy(tagm_hbm.at[b], tagm_v)

            def tot_body(t, tot):
                return tot + plsc.cumsum(tmm_v[pl.ds(t * L, L)])[L - 1]
            total = lax.fori_loop(0, NCH, tot_body, jnp.int32(0))

            def z_body(t, carry):
                tagbuf[pl.ds(t * L, L)] = jnp.zeros((L,), jnp.int32)
                return carry
            lax.fori_loop(0, NCH + 1, z_body, 0)

            def ch_body(t, carry):
                ctmm, ctok, ptr = carry
                v = tmm_v[pl.ds(t * L, L)]
                cs = plsc.cumsum(v) + ctmm
                kp = ((cs > 1) & (cs <= total - 1) & (v > 0)).astype(jnp.int32)
                tk = tok_v[pl.ds(t * L, L)] * kp
                tcs = plsc.cumsum(tk) + ctok
                keep_v[pl.ds(t * L, L)] = kp
                tokcs_v[pl.ds(t * L, L)] = tcs
                mi = (tagm_v[pl.ds(t * L, L)] > 0).astype(jnp.int32)
                mcs = plsc.cumsum(mi)
                idx = mcs - 1 + ptr
                plsc.store_scatter(tagbuf, [idx], tag_v[pl.ds(t * L, L)],
                                   mask=mi > 0)
                return (cs[L - 1], tcs[L - 1], ptr + mcs[L - 1])
            _, toksum, cnt = lax.fori_loop(
                0, NCH, ch_body,
                (jnp.int32(0), jnp.int32(0), jnp.int32(0)))

            pltpu.sync_copy(tagbuf.at[pl.ds(0, S)], v_hbm.at[pl.ds(b * S, S)])
            stg[...] = jnp.full((L,), cnt, jnp.int32)
            pltpu.sync_copy(stg, cnt_hbm.at[b])
            stg[...] = jnp.full((L,), toksum, jnp.int32)
            pltpu.sync_copy(stg, ts_hbm.at[b])

        plsc.subcore_barrier()

        @pl.when(c == 0)
        def _phase2():
            pltpu.sync_copy(cnt_hbm, clocal)
            pltpu.sync_copy(ts_hbm, tslocal)
            pltpu.sync_copy(v_hbm, vbuf)
            lanes = jnp.arange(L, dtype=jnp.int32)
            zeros16 = jnp.zeros((L,), jnp.int32)
            cvec = plsc.load_gather(clocal, [lanes, zeros16])
            tsvec = plsc.load_gather(tslocal, [lanes, zeros16])
            offarr[...] = plsc.cumsum(cvec) - cvec
            stg[...] = plsc.cumsum(tsvec) - tsvec
            off_tok_b = plsc.load_gather(stg, [jnp.full((L,), b, jnp.int32)])
            ovec = offarr[...]
            offs = [ovec[j] for j in range(1, B)]

            def ch2(t, carry):
                k = tokcs_v[pl.ds(t * L, L)] - 1 + off_tok_b
                r = jnp.zeros((L,), jnp.int32)
                for oj in offs:
                    r = r + (k >= oj).astype(jnp.int32)
                offr = plsc.load_gather(offarr, [r])
                lidx = jnp.clip(k - offr, 0, S - 1)
                vals = plsc.load_gather(vbuf, [r * S + lidx])
                tk = tok_v[pl.ds(t * L, L)] * keep_v[pl.ds(t * L, L)]
                sp = jnp.where(tk > 0, vals, jnp.int32(_IGNORE))
                tmm_v[pl.ds(t * L, L)] = sp
                return carry
            lax.fori_loop(0, NCH, ch2, 0)
            pltpu.sync_copy(tmm_v, sp_hbm.at[b])
            pltpu.sync_copy(keep_v, keep_hbm.at[b])

    return realign


def kernel(latent_states, attention_mask, token_mask, token_mask_mask,
           tag, tag_mask, W1, b1, W2, b2):
    B, S, D = latent_states.shape
    H = W1.shape[1]
    C = W2.shape[1]
    N = B * S
    G = N // _R

    Wd = tag.shape[1]
    tag_p = jnp.pad(tag, ((0, 0), (0, S - Wd)))
    tagm_p = jnp.pad(tag_mask, ((0, 0), (0, S - Wd)))
    sparsed_tag, keep, _, _, _ = _make_sc_realign(B, S)(
        token_mask, token_mask_mask, tag_p, tagm_p)

    xs = latent_states.reshape(N, D)
    st3 = sparsed_tag.reshape(G, 1, _R)
    mv3 = keep.reshape(G, 1, _R)

    def prev_map(i):
        return (jnp.maximum(i - 1, 0), 0, 0)

    out = pl.pallas_call(
        _tc_body,
        grid=(G,),
        in_specs=[
            pl.BlockSpec((_R, D), lambda i: (i, 0)),
            pl.BlockSpec((D, H), lambda i: (0, 0)),
            pl.BlockSpec((1, H), lambda i: (0, 0)),
            pl.BlockSpec((H, C), lambda i: (0, 0)),
            pl.BlockSpec((1, C), lambda i: (0, 0)),
            pl.BlockSpec((1, 1, _R), prev_map),
            pl.BlockSpec((1, 1, _R), prev_map),
            pl.BlockSpec((1, 1, _R), lambda i: (i, 0, 0)),
            pl.BlockSpec((1, 1, _R), lambda i: (i, 0, 0)),
        ],
        out_specs=pl.BlockSpec((8, 128), lambda i: (0, 0)),
        out_shape=jax.ShapeDtypeStruct((8, 128), jnp.float32),
        scratch_shapes=[pltpu.VMEM((_R, H), jnp.float32)],
    )(xs, W1, b1.reshape(1, H), W2, b2.reshape(1, C), st3, mv3, st3, mv3)

    nll_sum = out[0, 0]
    valid_cnt = out[0, 1]
    correct = out[0, 2]
    mvalid_cnt = out[0, 3]
    cost = _LAMBDA * nll_sum / jnp.maximum(valid_cnt, 1.0)
    acc = correct / jnp.maximum(mvalid_cnt, 1.0)
    return (cost, acc)


# speedup vs baseline: 1.0185x; 1.0185x over previous
import functools

import jax
import jax.numpy as jnp
from jax import lax
from jax.experimental import pallas as pl
from jax.experimental.pallas import tpu as pltpu
from jax.experimental.pallas import tpu_sc as plsc

_IGNORE = -100
_LAMBDA = 0.5
_R = 1024


def _tail_part(h, w2, b2, t, keep):
    logits = jnp.dot(h, w2, preferred_element_type=jnp.float32) + b2
    C = logits.shape[1]

    m = jnp.max(logits, axis=1)
    lse = m + jnp.log(jnp.sum(jnp.exp(logits - m[:, None]), axis=1))
    iota_c = lax.broadcasted_iota(jnp.int32, logits.shape, 1)
    xt = jnp.sum(jnp.where(iota_c == t[:, None], logits, 0.0), axis=1)
    validf = (t != _IGNORE).astype(jnp.float32)
    nll_sum = jnp.sum((lse - xt) * validf)
    valid_cnt = jnp.sum(validf)

    pred = jnp.min(jnp.where(logits == m[:, None], iota_c, C), axis=1)
    keepb = keep > 0
    correct = jnp.sum(((pred == t) & keepb).astype(jnp.float32))
    mvalid_cnt = jnp.sum(keepb.astype(jnp.float32))

    rows = lax.broadcasted_iota(jnp.int32, (8, 128), 0)
    cols = lax.broadcasted_iota(jnp.int32, (8, 128), 1)
    r0 = rows == 0
    return (jnp.where(r0 & (cols == 0), nll_sum, 0.0)
            + jnp.where(r0 & (cols == 1), valid_cnt, 0.0)
            + jnp.where(r0 & (cols == 2), correct, 0.0)
            + jnp.where(r0 & (cols == 3), mvalid_cnt, 0.0))


def _tc_body(x_ref, w1_ref, b1_ref, w2_ref, b2_ref, st_ref, mv_ref,
             stc_ref, mvc_ref, out_ref, h_ref):
    i = pl.program_id(0)
    G = pl.num_programs(0)

    @pl.when(i == 0)
    def _():
        h_ref[...] = jnp.zeros_like(h_ref)
        out_ref[...] = jnp.zeros_like(out_ref)

    part = _tail_part(h_ref[...], w2_ref[...], b2_ref[...],
                      st_ref[0, 0, :], mv_ref[0, 0, :])
    gate = jnp.where(i > 0, 1.0, 0.0).astype(jnp.float32)
    out_ref[...] += part * gate

    h_ref[...] = jnp.tanh(
        jnp.dot(x_ref[...], w1_ref[...], preferred_element_type=jnp.float32)
        + b1_ref[...])

    @pl.when(i == G - 1)
    def _():
        out_ref[...] += _tail_part(h_ref[...], w2_ref[...], b2_ref[...],
                                   stc_ref[0, 0, :], mvc_ref[0, 0, :])


def _make_sc_realign(B, S):
    L = 16
    NCH = S // L
    mesh = plsc.VectorSubcoreMesh(core_axis_name="c", subcore_axis_name="s")

    @functools.partial(
        pl.kernel, mesh=mesh,
        compiler_params=pltpu.CompilerParams(needs_layout_passes=False),
        out_type=(
            jax.ShapeDtypeStruct((B, S), jnp.int32),
            jax.ShapeDtypeStruct((B, S), jnp.int32),
            jax.ShapeDtypeStruct((B * S,), jnp.int32),
            jax.ShapeDtypeStruct((B, L), jnp.int32),
            jax.ShapeDtypeStruct((B, L), jnp.int32),
        ),
        scratch_types=[
            pltpu.VMEM((S,), jnp.int32),
            pltpu.VMEM((S,), jnp.int32),
            pltpu.VMEM((S,), jnp.int32),
            pltpu.VMEM((S,), jnp.int32),
            pltpu.VMEM((S,), jnp.int32),
            pltpu.VMEM((S,), jnp.int32),
            pltpu.VMEM((S + L,), jnp.int32),
            pltpu.VMEM((B * S,), jnp.int32),
            pltpu.VMEM((B, L), jnp.int32),
            pltpu.VMEM((B, L), jnp.int32),
            pltpu.VMEM((L,), jnp.int32),
            pltpu.VMEM((L,), jnp.int32),
            pltpu.SemaphoreType.DMA,
        ])
    def realign(tm_hbm, tmm_hbm, tag_hbm, tagm_hbm,
                sp_hbm, keep_hbm, v_hbm, cnt_hbm, ts_hbm,
                tmm_v, tok_v, tag_v, tagm_v, keep_v, tokcs_v, tagbuf,
                vbuf, clocal, tslocal, offarr, stg, sem):
        c = lax.axis_index("c")
        b = lax.axis_index("s")

        @pl.when(c == 0)
        def _phase1():
            cps = [pltpu.async_copy(tmm_hbm.at[b], tmm_v, sem),
                   pltpu.async_copy(tm_hbm.at[b], tok_v, sem),
                   pltpu.async_copy(tag_hbm.at[b], tag_v, sem),
                   pltpu.async_copy(tagm_hbm.at[b], tagm_v, sem)]
            cps[0].wait()

            def tot_body(t, tot):
                tagbuf[pl.ds(t * L, L)] = jnp.zeros((L,), jnp.int32)
                return tot + plsc.cumsum(tmm_v[pl.ds(t * L, L)])[L - 1]
            total = lax.fori_loop(0, NCH, tot_body, jnp.int32(0), unroll=2)
            tagbuf[pl.ds(NCH * L, L)] = jnp.zeros((L,), jnp.int32)
            cps[1].wait()
            cps[2].wait()
            cps[3].wait()

            def ch_body(t, carry):
                ctmm, ctok, ptr = carry
                v = tmm_v[pl.ds(t * L, L)]
                cs = plsc.cumsum(v) + ctmm
                kp = ((cs > 1) & (cs <= total - 1) & (v > 0)).astype(jnp.int32)
                tk = tok_v[pl.ds(t * L, L)] * kp
                tcs = plsc.cumsum(tk) + ctok
                keep_v[pl.ds(t * L, L)] = kp
                tokcs_v[pl.ds(t * L, L)] = tcs
                mi = (tagm_v[pl.ds(t * L, L)] > 0).astype(jnp.int32)
                mcs = plsc.cumsum(mi)
                idx = mcs - 1 + ptr
                plsc.store_scatter(tagbuf, [idx], tag_v[pl.ds(t * L, L)],
                                   mask=mi > 0)
                return (cs[L - 1], tcs[L - 1], ptr + mcs[L - 1])
            _, toksum, cnt = lax.fori_loop(
                0, NCH, ch_body,
                (jnp.int32(0), jnp.int32(0), jnp.int32(0)), unroll=2)

            pltpu.sync_copy(tagbuf.at[pl.ds(0, S)], v_hbm.at[pl.ds(b * S, S)])
            stg[...] = jnp.full((L,), cnt, jnp.int32)
            pltpu.sync_copy(stg, cnt_hbm.at[b])
            stg[...] = jnp.full((L,), toksum, jnp.int32)
            pltpu.sync_copy(stg, ts_hbm.at[b])

        plsc.subcore_barrier()

        @pl.when(c == 0)
        def _phase2():
            vcp = pltpu.async_copy(v_hbm, vbuf, sem)
            pltpu.sync_copy(cnt_hbm, clocal)
            pltpu.sync_copy(ts_hbm, tslocal)
            lanes = jnp.arange(L, dtype=jnp.int32)
            zeros16 = jnp.zeros((L,), jnp.int32)
            cvec = plsc.load_gather(clocal, [lanes, zeros16])
            tsvec = plsc.load_gather(tslocal, [lanes, zeros16])
            offarr[...] = plsc.cumsum(cvec) - cvec
            stg[...] = plsc.cumsum(tsvec) - tsvec
            off_tok_b = plsc.load_gather(stg, [jnp.full((L,), b, jnp.int32)])
            ovec = offarr[...]
            offs = [ovec[j] for j in range(1, B)]
            vcp.wait()

            def ch2(t, carry):
                k = tokcs_v[pl.ds(t * L, L)] - 1 + off_tok_b
                r = jnp.zeros((L,), jnp.int32)
                for oj in offs:
                    r = r + (k >= oj).astype(jnp.int32)
                offr = plsc.load_gather(offarr, [r])
                lidx = jnp.clip(k - offr, 0, S - 1)
                vals = plsc.load_gather(vbuf, [r * S + lidx])
                tk = tok_v[pl.ds(t * L, L)] * keep_v[pl.ds(t * L, L)]
                sp = jnp.where(tk > 0, vals, jnp.int32(_IGNORE))
                tmm_v[pl.ds(t * L, L)] = sp
                return carry
            lax.fori_loop(0, NCH, ch2, 0, unroll=2)
            pltpu.sync_copy(tmm_v, sp_hbm.at[b])
            pltpu.sync_copy(keep_v, keep_hbm.at[b])

    return realign


def kernel(latent_states, attention_mask, token_mask, token_mask_mask,
           tag, tag_mask, W1, b1, W2, b2):
    B, S, D = latent_states.shape
    H = W1.shape[1]
    C = W2.shape[1]
    N = B * S
    G = N // _R

    Wd = tag.shape[1]
    tag_p = jnp.pad(tag, ((0, 0), (0, S - Wd)))
    tagm_p = jnp.pad(tag_mask, ((0, 0), (0, S - Wd)))
    sparsed_tag, keep, _, _, _ = _make_sc_realign(B, S)(
        token_mask, token_mask_mask, tag_p, tagm_p)

    xs = latent_states.reshape(N, D)
    st3 = sparsed_tag.reshape(G, 1, _R)
    mv3 = keep.reshape(G, 1, _R)

    def prev_map(i):
        return (jnp.maximum(i - 1, 0), 0, 0)

    out = pl.pallas_call(
        _tc_body,
        grid=(G,),
        in_specs=[
            pl.BlockSpec((_R, D), lambda i: (i, 0)),
            pl.BlockSpec((D, H), lambda i: (0, 0)),
            pl.BlockSpec((1, H), lambda i: (0, 0)),
            pl.BlockSpec((H, C), lambda i: (0, 0)),
            pl.BlockSpec((1, C), lambda i: (0, 0)),
            pl.BlockSpec((1, 1, _R), prev_map),
            pl.BlockSpec((1, 1, _R), prev_map),
            pl.BlockSpec((1, 1, _R), lambda i: (i, 0, 0)),
            pl.BlockSpec((1, 1, _R), lambda i: (i, 0, 0)),
        ],
        out_specs=pl.BlockSpec((8, 128), lambda i: (0, 0)),
        out_shape=jax.ShapeDtypeStruct((8, 128), jnp.float32),
        scratch_shapes=[pltpu.VMEM((_R, H), jnp.float32)],
    )(xs, W1, b1.reshape(1, H), W2, b2.reshape(1, C), st3, mv3, st3, mv3)

    nll_sum = out[0, 0]
    valid_cnt = out[0, 1]
    correct = out[0, 2]
    mvalid_cnt = out[0, 3]
    cost = _LAMBDA * nll_sum / jnp.maximum(valid_cnt, 1.0)
    acc = correct / jnp.maximum(mvalid_cnt, 1.0)
    return (cost, acc)


# speedup vs baseline: 1.0228x; 1.0043x over previous
import functools

import jax
import jax.numpy as jnp
from jax import lax
from jax.experimental import pallas as pl
from jax.experimental.pallas import tpu as pltpu
from jax.experimental.pallas import tpu_sc as plsc

_IGNORE = -100
_LAMBDA = 0.5
_R = 1024


def _tail_part(hT, w2, b2c, t, keep):
    logitsT = lax.dot_general(w2, hT, (((0,), (0,)), ((), ())),
                              preferred_element_type=jnp.float32) + b2c
    C = logitsT.shape[0]

    m = jnp.max(logitsT, axis=0)
    lse = m + jnp.log(jnp.sum(jnp.exp(logitsT - m[None, :]), axis=0))
    iota_c = lax.broadcasted_iota(jnp.int32, logitsT.shape, 0)
    xt = jnp.sum(jnp.where(iota_c == t[None, :], logitsT, 0.0), axis=0)
    validf = (t != _IGNORE).astype(jnp.float32)
    nll_sum = jnp.sum((lse - xt) * validf)
    valid_cnt = jnp.sum(validf)

    pred = jnp.min(jnp.where(logitsT == m[None, :], iota_c, C), axis=0)
    keepb = keep > 0
    correct = jnp.sum(((pred == t) & keepb).astype(jnp.float32))
    mvalid_cnt = jnp.sum(keepb.astype(jnp.float32))

    rows = lax.broadcasted_iota(jnp.int32, (8, 128), 0)
    cols = lax.broadcasted_iota(jnp.int32, (8, 128), 1)
    r0 = rows == 0
    return (jnp.where(r0 & (cols == 0), nll_sum, 0.0)
            + jnp.where(r0 & (cols == 1), valid_cnt, 0.0)
            + jnp.where(r0 & (cols == 2), correct, 0.0)
            + jnp.where(r0 & (cols == 3), mvalid_cnt, 0.0))


def _tc_body(x_ref, w1_ref, b1_ref, w2_ref, b2_ref, st_ref, mv_ref,
             stc_ref, mvc_ref, out_ref, h_ref):
    i = pl.program_id(0)
    G = pl.num_programs(0)

    @pl.when(i == 0)
    def _():
        h_ref[...] = jnp.zeros_like(h_ref)
        out_ref[...] = jnp.zeros_like(out_ref)

    part = _tail_part(h_ref[...], w2_ref[...], b2_ref[...],
                      st_ref[0, 0, :], mv_ref[0, 0, :])
    gate = jnp.where(i > 0, 1.0, 0.0).astype(jnp.float32)
    out_ref[...] += part * gate

    h_ref[...] = jnp.tanh(
        lax.dot_general(w1_ref[...], x_ref[...], (((0,), (1,)), ((), ())),
                        preferred_element_type=jnp.float32)
        + b1_ref[...])

    @pl.when(i == G - 1)
    def _():
        out_ref[...] += _tail_part(h_ref[...], w2_ref[...], b2_ref[...],
                                   stc_ref[0, 0, :], mvc_ref[0, 0, :])


def _make_sc_realign(B, S):
    L = 16
    NCH = S // L
    mesh = plsc.VectorSubcoreMesh(core_axis_name="c", subcore_axis_name="s")

    @functools.partial(
        pl.kernel, mesh=mesh,
        compiler_params=pltpu.CompilerParams(needs_layout_passes=False),
        out_type=(
            jax.ShapeDtypeStruct((B, S), jnp.int32),
            jax.ShapeDtypeStruct((B, S), jnp.int32),
            jax.ShapeDtypeStruct((B * S,), jnp.int32),
            jax.ShapeDtypeStruct((B, L), jnp.int32),
            jax.ShapeDtypeStruct((B, L), jnp.int32),
        ),
        scratch_types=[
            pltpu.VMEM((S,), jnp.int32),
            pltpu.VMEM((S,), jnp.int32),
            pltpu.VMEM((S,), jnp.int32),
            pltpu.VMEM((S,), jnp.int32),
            pltpu.VMEM((S,), jnp.int32),
            pltpu.VMEM((S,), jnp.int32),
            pltpu.VMEM((S + L,), jnp.int32),
            pltpu.VMEM((B * S,), jnp.int32),
            pltpu.VMEM((B, L), jnp.int32),
            pltpu.VMEM((B, L), jnp.int32),
            pltpu.VMEM((L,), jnp.int32),
            pltpu.VMEM((L,), jnp.int32),
            pltpu.SemaphoreType.DMA,
        ])
    def realign(tm_hbm, tmm_hbm, tag_hbm, tagm_hbm,
                sp_hbm, keep_hbm, v_hbm, cnt_hbm, ts_hbm,
                tmm_v, tok_v, tag_v, tagm_v, keep_v, tokcs_v, tagbuf,
                vbuf, clocal, tslocal, offarr, stg, sem):
        c = lax.axis_index("c")
        b = lax.axis_index("s")

        @pl.when(c == 0)
        def _phase1():
            cps = [pltpu.async_copy(tmm_hbm.at[b], tmm_v, sem),
                   pltpu.async_copy(tm_hbm.at[b], tok_v, sem),
                   pltpu.async_copy(tag_hbm.at[b], tag_v, sem),
                   pltpu.async_copy(tagm_hbm.at[b], tagm_v, sem)]
            cps[0].wait()

            def tot_body(t, tot):
                tagbuf[pl.ds(t * L, L)] = jnp.zeros((L,), jnp.int32)
                return tot + plsc.cumsum(tmm_v[pl.ds(t * L, L)])[L - 1]
            total = lax.fori_loop(0, NCH, tot_body, jnp.int32(0), unroll=2)
            tagbuf[pl.ds(NCH * L, L)] = jnp.zeros((L,), jnp.int32)
            cps[1].wait()
            cps[2].wait()
            cps[3].wait()

            def ch_body(t, carry):
                ctmm, ctok, ptr = carry
                v = tmm_v[pl.ds(t * L, L)]
                cs = plsc.cumsum(v) + ctmm
                kp = ((cs > 1) & (cs <= total - 1) & (v > 0)).astype(jnp.int32)
                tk = tok_v[pl.ds(t * L, L)] * kp
                tcs = plsc.cumsum(tk) + ctok
                keep_v[pl.ds(t * L, L)] = kp
                tokcs_v[pl.ds(t * L, L)] = tcs
                mi = (tagm_v[pl.ds(t * L, L)] > 0).astype(jnp.int32)
                mcs = plsc.cumsum(mi)
                idx = mcs - 1 + ptr
                plsc.store_scatter(tagbuf, [idx], tag_v[pl.ds(t * L, L)],
                                   mask=mi > 0)
                return (cs[L - 1], tcs[L - 1], ptr + mcs[L - 1])
            _, toksum, cnt = lax.fori_loop(
                0, NCH, ch_body,
                (jnp.int32(0), jnp.int32(0), jnp.int32(0)), unroll=2)

            pltpu.sync_copy(tagbuf.at[pl.ds(0, S)], v_hbm.at[pl.ds(b * S, S)])
            stg[...] = jnp.full((L,), cnt, jnp.int32)
            pltpu.sync_copy(stg, cnt_hbm.at[b])
            stg[...] = jnp.full((L,), toksum, jnp.int32)
            pltpu.sync_copy(stg, ts_hbm.at[b])

        plsc.subcore_barrier()

        @pl.when(c == 0)
        def _phase2():
            vcp = pltpu.async_copy(v_hbm, vbuf, sem)
            pltpu.sync_copy(cnt_hbm, clocal)
            pltpu.sync_copy(ts_hbm, tslocal)
            lanes = jnp.arange(L, dtype=jnp.int32)
            zeros16 = jnp.zeros((L,), jnp.int32)
            cvec = plsc.load_gather(clocal, [lanes, zeros16])
            tsvec = plsc.load_gather(tslocal, [lanes, zeros16])
            offarr[...] = plsc.cumsum(cvec) - cvec
            stg[...] = plsc.cumsum(tsvec) - tsvec
            off_tok_b = plsc.load_gather(stg, [jnp.full((L,), b, jnp.int32)])
            ovec = offarr[...]
            offs = [ovec[j] for j in range(1, B)]
            vcp.wait()

            def ch2(t, carry):
                k = tokcs_v[pl.ds(t * L, L)] - 1 + off_tok_b
                r = jnp.zeros((L,), jnp.int32)
                for oj in offs:
                    r = r + (k >= oj).astype(jnp.int32)
                offr = plsc.load_gather(offarr, [r])
                lidx = jnp.clip(k - offr, 0, S - 1)
                vals = plsc.load_gather(vbuf, [r * S + lidx])
                tk = tok_v[pl.ds(t * L, L)] * keep_v[pl.ds(t * L, L)]
                sp = jnp.where(tk > 0, vals, jnp.int32(_IGNORE))
                tmm_v[pl.ds(t * L, L)] = sp
                return carry
            lax.fori_loop(0, NCH, ch2, 0, unroll=2)
            pltpu.sync_copy(tmm_v, sp_hbm.at[b])
            pltpu.sync_copy(keep_v, keep_hbm.at[b])

    return realign


def kernel(latent_states, attention_mask, token_mask, token_mask_mask,
           tag, tag_mask, W1, b1, W2, b2):
    B, S, D = latent_states.shape
    H = W1.shape[1]
    C = W2.shape[1]
    N = B * S
    G = N // _R

    Wd = tag.shape[1]
    tag_p = jnp.pad(tag, ((0, 0), (0, S - Wd)))
    tagm_p = jnp.pad(tag_mask, ((0, 0), (0, S - Wd)))
    sparsed_tag, keep, _, _, _ = _make_sc_realign(B, S)(
        token_mask, token_mask_mask, tag_p, tagm_p)

    xs = latent_states.reshape(N, D)
    st3 = sparsed_tag.reshape(G, 1, _R)
    mv3 = keep.reshape(G, 1, _R)

    def prev_map(i):
        return (jnp.maximum(i - 1, 0), 0, 0)

    out = pl.pallas_call(
        _tc_body,
        grid=(G,),
        in_specs=[
            pl.BlockSpec((_R, D), lambda i: (i, 0)),
            pl.BlockSpec((D, H), lambda i: (0, 0)),
            pl.BlockSpec((H, 1), lambda i: (0, 0)),
            pl.BlockSpec((H, C), lambda i: (0, 0)),
            pl.BlockSpec((C, 1), lambda i: (0, 0)),
            pl.BlockSpec((1, 1, _R), prev_map),
            pl.BlockSpec((1, 1, _R), prev_map),
            pl.BlockSpec((1, 1, _R), lambda i: (i, 0, 0)),
            pl.BlockSpec((1, 1, _R), lambda i: (i, 0, 0)),
        ],
        out_specs=pl.BlockSpec((8, 128), lambda i: (0, 0)),
        out_shape=jax.ShapeDtypeStruct((8, 128), jnp.float32),
        scratch_shapes=[pltpu.VMEM((H, _R), jnp.float32)],
    )(xs, W1, b1.reshape(H, 1), W2, b2.reshape(C, 1), st3, mv3, st3, mv3)

    nll_sum = out[0, 0]
    valid_cnt = out[0, 1]
    correct = out[0, 2]
    mvalid_cnt = out[0, 3]
    cost = _LAMBDA * nll_sum / jnp.maximum(valid_cnt, 1.0)
    acc = correct / jnp.maximum(mvalid_cnt, 1.0)
    return (cost, acc)
